# Initial kernel scaffold; baseline (speedup 1.0000x reference)
import jax
import jax.numpy as jnp
from jax.experimental import pallas as pl


def kernel(x, edge_attr, params, edge_index, batch):
    raise NotImplementedError("write your pallas kernel here")


# speedup vs baseline: 1.2514x; 1.2514x over previous
import functools

import jax
import jax.numpy as jnp
from jax import lax
from jax.experimental import pallas as pl
from jax.experimental.pallas import tpu as pltpu
from jax.experimental.pallas import tpu_sc as plsc

_N = 10000
_E = 16000
_HID = 32
_NCORE = 2
_NSUB = 16
_NW = _NCORE * _NSUB
_EPW = 512
_EPAD = _NW * _EPW
_CHK = 128
_NCHK = _EPW // _CHK
_RPT = _N // _NSUB
_DIMS = ((32, 32), (32, 32), (32, 64), (64, 64), (64, 64))

_mesh = plsc.VectorSubcoreMesh(core_axis_name="c", subcore_axis_name="s")


@functools.cache
def _gather_fn(ci):
    @functools.partial(
        pl.kernel,
        out_type=jax.ShapeDtypeStruct((_EPAD, ci), jnp.float32),
        mesh=_mesh,
        scratch_types=[
            pltpu.VMEM((_NCHK, _CHK), jnp.int32),
            pltpu.VMEM((_EPW, ci), jnp.float32),
            pltpu.SemaphoreType.DMA,
        ],
        compiler_params=pltpu.CompilerParams(use_tc_tiling_on_sc=False),
    )
    def gat(table, idx, out, idx_v, rows_v, sem):
        w = lax.axis_index("s") * _NCORE + lax.axis_index("c")
        pltpu.sync_copy(idx.at[w], idx_v)
        descs = [
            pltpu.async_copy(
                table.at[idx_v.at[j]], rows_v.at[pl.ds(j * _CHK, _CHK)], sem
            )
            for j in range(_NCHK)
        ]
        for d in descs:
            d.wait()
        pltpu.sync_copy(rows_v, out.at[pl.ds(w * _EPW, _EPW)])

    return gat


@functools.cache
def _scatter_fn(co):
    @functools.partial(
        pl.kernel,
        out_type=jax.ShapeDtypeStruct((_NCORE, _N, co), jnp.float32),
        mesh=_mesh,
        scratch_types=[
            pltpu.VMEM((_NCHK, _CHK), jnp.int32),
            pltpu.VMEM((_EPW, co), jnp.float32),
            pltpu.VMEM_SHARED((_N, co), jnp.float32),
            pltpu.SemaphoreType.DMA,
        ],
        compiler_params=pltpu.CompilerParams(use_tc_tiling_on_sc=False),
    )
    def sca(msg, dstidx, zeros, out, idx_v, msg_v, shared, sem):
        c = lax.axis_index("c")
        s = lax.axis_index("s")
        w = s * _NCORE + c
        pltpu.sync_copy(dstidx.at[w], idx_v)
        pltpu.sync_copy(msg.at[pl.ds(w * _EPW, _EPW)], msg_v)
        pltpu.sync_copy(
            zeros.at[pl.ds(s * _RPT, _RPT)], shared.at[pl.ds(s * _RPT, _RPT)]
        )
        plsc.subcore_barrier()
        for j in range(_NCHK):
            pltpu.sync_copy(
                msg_v.at[pl.ds(j * _CHK, _CHK)],
                shared.at[idx_v.at[j]],
                add=True,
            )
        plsc.subcore_barrier()
        pltpu.sync_copy(
            shared.at[pl.ds(s * _RPT, _RPT)],
            out.at[c].at[pl.ds(s * _RPT, _RPT)],
        )

    return sca


@functools.cache
def _msg_fn(ci, co):
    be = 512
    grid = _EPAD // be

    def body(xj_ref, ea_ref, w1_ref, b1_ref, w2_ref, b2_ref, o_ref):
        i = pl.program_id(0)
        rows = lax.broadcasted_iota(jnp.int32, (be, 1), 0) + i * be
        valid = rows < _E
        xj = jnp.where(valid, xj_ref[...], 0.0)
        hid = jnp.maximum(ea_ref[...] * w1_ref[...] + b1_ref[...], 0.0)
        hid = jnp.where(valid, hid, 0.0)
        t = lax.dot_general(
            xj, w2_ref[...], (((1,), (0,)), ((), ())),
            preferred_element_type=jnp.float32,
        )
        msg = lax.dot_general(
            xj, b2_ref[...], (((1,), (0,)), ((), ())),
            preferred_element_type=jnp.float32,
        )
        for h in range(_HID):
            msg = msg + hid[:, h:h + 1] * t[:, h * co:(h + 1) * co]
        o_ref[...] = msg

    return pl.pallas_call(
        body,
        grid=(grid,),
        in_specs=[
            pl.BlockSpec((be, ci), lambda i: (i, 0)),
            pl.BlockSpec((be, 1), lambda i: (i, 0)),
            pl.BlockSpec((1, _HID), lambda i: (0, 0)),
            pl.BlockSpec((1, _HID), lambda i: (0, 0)),
            pl.BlockSpec((ci, _HID * co), lambda i: (0, 0)),
            pl.BlockSpec((ci, co), lambda i: (0, 0)),
        ],
        out_specs=pl.BlockSpec((be, co), lambda i: (i, 0)),
        out_shape=jax.ShapeDtypeStruct((_EPAD, co), jnp.float32),
    )


@functools.cache
def _update_fn(ci, co):
    def body(p0_ref, p1_ref, h_ref, root_ref, bias_ref, o_ref):
        v = (
            p0_ref[...]
            + p1_ref[...]
            + lax.dot_general(
                h_ref[...], root_ref[...], (((1,), (0,)), ((), ())),
                preferred_element_type=jnp.float32,
            )
            + bias_ref[...]
        )
        o_ref[...] = jnp.maximum(v, 0.0)

    return pl.pallas_call(
        body,
        out_shape=jax.ShapeDtypeStruct((_N, co), jnp.float32),
    )


@functools.cache
def _final_fn(ci, co, nc):
    def body(p0_ref, p1_ref, h_ref, root_ref, bias_ref, fcw_ref, fcb_ref,
             o_ref):
        hn = (
            p0_ref[...]
            + p1_ref[...]
            + lax.dot_general(
                h_ref[...], root_ref[...], (((1,), (0,)), ((), ())),
                preferred_element_type=jnp.float32,
            )
            + bias_ref[...]
        )
        emb = jnp.max(hn, axis=0, keepdims=True)
        o_ref[...] = (
            lax.dot_general(
                jnp.maximum(emb, 0.0), fcw_ref[...],
                (((1,), (0,)), ((), ())),
                preferred_element_type=jnp.float32,
            )
            + fcb_ref[...]
        )

    return pl.pallas_call(
        body,
        out_shape=jax.ShapeDtypeStruct((1, nc), jnp.float32),
    )


def kernel(x, edge_attr, params, edge_index, batch):
    pad = _EPAD - _E
    srcp = jnp.concatenate(
        [edge_index[0], jnp.zeros((pad,), jnp.int32)]
    ).reshape(_NW, _NCHK, _CHK)
    dstp = jnp.concatenate(
        [edge_index[1], jnp.zeros((pad,), jnp.int32)]
    ).reshape(_NW, _NCHK, _CHK)
    eap = jnp.concatenate([edge_attr, jnp.zeros((pad, 1), jnp.float32)])
    zeros64 = jnp.zeros((_N, 64), jnp.float32)

    h = x
    out = None
    for i, (ci, co) in enumerate(_DIMS):
        p = params["l%d" % i]
        w2c = (
            p["w2"].reshape(_HID, ci, co).transpose(1, 0, 2).reshape(ci, _HID * co)
        )
        b2m = p["b2"].reshape(ci, co)
        xj = _gather_fn(ci)(h, srcp)
        msg = _msg_fn(ci, co)(
            xj, eap, p["w1"].reshape(1, _HID), p["b1"].reshape(1, _HID),
            w2c, b2m,
        )
        part = _scatter_fn(co)(msg, dstp, zeros64[:, :co])
        if i + 1 < len(_DIMS):
            h = _update_fn(ci, co)(
                part[0], part[1], h, p["root"], p["bias"].reshape(1, co)
            )
        else:
            nc = params["fc_w"].shape[1]
            out = _final_fn(ci, co, nc)(
                part[0], part[1], h, p["root"], p["bias"].reshape(1, co),
                params["fc_w"], params["fc_b"].reshape(1, nc),
            )
    return out
